# Initial kernel scaffold; baseline (speedup 1.0000x reference)
import jax
import jax.numpy as jnp
from jax.experimental import pallas as pl


def kernel(X, A, W_rel, b_rel, W_root, W_mlp, b_mlp):
    raise NotImplementedError("write your pallas kernel here")


# speedup vs baseline: 802.4793x; 802.4793x over previous
import jax
import jax.numpy as jnp
from jax.experimental import pallas as pl

_N, _T, _F, _K = 1024, 128, 128, 32


def _mcc_kernel(x_ref, a_ref, wrel_ref, brel_ref, wroot_ref, wmlp_ref,
                bmlp_ref, s_ref, mc_ref, lo_ref):
    a = a_ref[...]
    x = x_ref[...]

    deg = jnp.sum(a, axis=1, keepdims=True)
    rs_col = jax.lax.rsqrt(deg)
    rs_row = jnp.transpose(rs_col)
    a_n = (a * rs_col) * rs_row
    mask = (a_n != 0).astype(jnp.float32)

    mu = jnp.mean(x)
    var = jnp.mean((x - mu) ** 2)
    xn = (x - mu) * jax.lax.rsqrt(var + 1e-5)

    aggr = jax.lax.dot_general(mask, xn, (((0,), (0,)), ((), ())),
                               preferred_element_type=jnp.float32)
    xg = (jax.lax.dot_general(aggr, wrel_ref[...], (((1,), (1,)), ((), ())),
                              preferred_element_type=jnp.float32)
          + brel_ref[...]
          + jax.lax.dot_general(xn, wroot_ref[...], (((1,), (1,)), ((), ())),
                                preferred_element_type=jnp.float32))
    s_logits = jax.lax.dot_general(xg, wmlp_ref[...], (((1,), (1,)), ((), ())),
                                   preferred_element_type=jnp.float32)
    s_logits = s_logits + bmlp_ref[...]
    s_ref[...] = s_logits

    s = jax.nn.softmax(s_logits, axis=-1)
    an_s = jax.lax.dot_general(a_n, s, (((1,), (0,)), ((), ())),
                               preferred_element_type=jnp.float32)
    mincut_num = jnp.sum(s * an_s)
    d_flat = jnp.sum(a_n, axis=1, keepdims=True)
    mincut_den = jnp.sum(d_flat * jnp.sum(s * s, axis=1, keepdims=True))
    mc_ref[...] = (-(mincut_num / mincut_den)).reshape(1, 1)

    ss = jax.lax.dot_general(s, s, (((0,), (0,)), ((), ())),
                             preferred_element_type=jnp.float32)
    n_ss = jnp.sqrt(jnp.sum(ss * ss))
    ii = jax.lax.broadcasted_iota(jnp.int32, (_K, _K), 0)
    jj = jax.lax.broadcasted_iota(jnp.int32, (_K, _K), 1)
    eye = (ii == jj).astype(jnp.float32)
    diff = ss / n_ss - eye / jnp.sqrt(jnp.float32(_K))
    lo_ref[...] = jnp.sqrt(jnp.sum(diff * diff)).reshape(1, 1)


def kernel(X, A, W_rel, b_rel, W_root, W_mlp, b_mlp):
    out_shape = (
        jax.ShapeDtypeStruct((_N, _K), jnp.float32),
        jax.ShapeDtypeStruct((1, 1), jnp.float32),
        jax.ShapeDtypeStruct((1, 1), jnp.float32),
    )
    S, mc, lo = pl.pallas_call(_mcc_kernel, out_shape=out_shape)(
        X, A, W_rel, b_rel.reshape(1, _F), W_root, W_mlp,
        b_mlp.reshape(1, _K))
    return (S, mc[0, 0], lo[0, 0])


# speedup vs baseline: 835.2313x; 1.0408x over previous
import jax
import jax.numpy as jnp
from jax.experimental import pallas as pl

_N, _T, _F, _K = 1024, 128, 128, 32


def _mcc_kernel(x_ref, a_ref, wrel_ref, brel_ref, wroot_ref, wmlp_ref,
                bmlp_ref, s_ref, mc_ref, lo_ref):
    a = a_ref[...]
    x = x_ref[...]

    deg = jnp.sum(a, axis=1, keepdims=True)
    rs_col = jax.lax.rsqrt(deg)
    mask = (a != 0).astype(jnp.float32)

    mu = jnp.mean(x)
    var = jnp.mean((x - mu) ** 2)
    xn = (x - mu) * jax.lax.rsqrt(var + 1e-5)

    aggr = jax.lax.dot_general(mask, xn, (((0,), (0,)), ((), ())),
                               preferred_element_type=jnp.float32)

    w_rel2 = jax.lax.dot_general(wmlp_ref[...], wrel_ref[...],
                                 (((1,), (0,)), ((), ())),
                                 preferred_element_type=jnp.float32)
    w_root2 = jax.lax.dot_general(wmlp_ref[...], wroot_ref[...],
                                  (((1,), (0,)), ((), ())),
                                  preferred_element_type=jnp.float32)
    b2 = jax.lax.dot_general(brel_ref[...], wmlp_ref[...],
                             (((1,), (1,)), ((), ())),
                             preferred_element_type=jnp.float32)
    s_logits = (jax.lax.dot_general(aggr, w_rel2, (((1,), (1,)), ((), ())),
                                    preferred_element_type=jnp.float32)
                + jax.lax.dot_general(xn, w_root2, (((1,), (1,)), ((), ())),
                                      preferred_element_type=jnp.float32)
                + b2 + bmlp_ref[...])
    s_ref[...] = s_logits

    s = jax.nn.softmax(s_logits, axis=-1)
    u = s * rs_col
    urs = jnp.concatenate([u, rs_col], axis=1)
    a_urs = jax.lax.dot_general(a, urs, (((1,), (0,)), ((), ())),
                                preferred_element_type=jnp.float32)
    au = a_urs[:, :_K]
    d_flat = rs_col * a_urs[:, _K:]
    mincut_num = jnp.sum(u * au)
    mincut_den = jnp.sum(d_flat * jnp.sum(s * s, axis=1, keepdims=True))
    mc_ref[...] = (-(mincut_num / mincut_den)).reshape(1, 1)

    ss = jax.lax.dot_general(s, s, (((0,), (0,)), ((), ())),
                             preferred_element_type=jnp.float32)
    n_ss = jnp.sqrt(jnp.sum(ss * ss))
    ii = jax.lax.broadcasted_iota(jnp.int32, (_K, _K), 0)
    jj = jax.lax.broadcasted_iota(jnp.int32, (_K, _K), 1)
    eye = (ii == jj).astype(jnp.float32)
    diff = ss / n_ss - eye / jnp.sqrt(jnp.float32(_K))
    lo_ref[...] = jnp.sqrt(jnp.sum(diff * diff)).reshape(1, 1)


def kernel(X, A, W_rel, b_rel, W_root, W_mlp, b_mlp):
    out_shape = (
        jax.ShapeDtypeStruct((_N, _K), jnp.float32),
        jax.ShapeDtypeStruct((1, 1), jnp.float32),
        jax.ShapeDtypeStruct((1, 1), jnp.float32),
    )
    S, mc, lo = pl.pallas_call(_mcc_kernel, out_shape=out_shape)(
        X, A, W_rel, b_rel.reshape(1, _F), W_root, W_mlp,
        b_mlp.reshape(1, _K))
    return (S, mc[0, 0], lo[0, 0])
